# baseline (speedup 1.0000x reference)
import functools

import jax
import jax.numpy as jnp
from jax import lax
from jax.experimental import pallas as pl
from jax.experimental.pallas import tpu as pltpu
from jax.experimental.pallas import tpu_sc as plsc

D = 64
NNEG = 5
NC = 2
NS = 16
L = 16
NW = NC * NS
NV = D // L
NT = 1 + NNEG


@functools.lru_cache(maxsize=None)
def _make_sc_partials(B: int):
    assert B % NW == 0
    bpw = B // NW
    mesh = plsc.VectorSubcoreMesh(core_axis_name="c", subcore_axis_name="s")

    @functools.partial(
        pl.kernel,
        out_type=jax.ShapeDtypeStruct((NW, NT * L), jnp.float32),
        mesh=mesh,
        compiler_params=pltpu.CompilerParams(use_tc_tiling_on_sc=False),
        scratch_types=[
            pltpu.VMEM((bpw,), jnp.int32),
            pltpu.VMEM((bpw,), jnp.int32),
            pltpu.VMEM((NNEG, bpw), jnp.int32),
            pltpu.VMEM((bpw, D), jnp.float32),
            pltpu.VMEM((bpw, D), jnp.float32),
            pltpu.VMEM((NNEG, bpw, D), jnp.float32),
            pltpu.VMEM((NT * L,), jnp.float32),
            pltpu.SemaphoreType.DMA,
        ],
    )
    def sc_partials(iw_hbm, cw_hbm, nw_hbm, u_hbm, v_hbm, out_hbm,
                    idx_u, idx_c, idx_n, u_rows, v_rows, n_rows, pout, sem):
        wid = lax.axis_index("s") * NC + lax.axis_index("c")
        base = wid * bpw

        pltpu.sync_copy(iw_hbm.at[pl.ds(base, bpw)], idx_u)
        pltpu.sync_copy(cw_hbm.at[pl.ds(base, bpw)], idx_c)
        for n in range(NNEG):
            pltpu.sync_copy(nw_hbm.at[pl.ds(n * B + base, bpw)], idx_n.at[n])

        copies = [
            pltpu.async_copy(u_hbm.at[idx_u], u_rows, sem),
            pltpu.async_copy(v_hbm.at[idx_c], v_rows, sem),
        ]
        for n in range(NNEG):
            copies.append(
                pltpu.async_copy(v_hbm.at[idx_n.at[n]], n_rows.at[n], sem))
        for cp in copies:
            cp.wait()

        def body(i, accs):
            accs = list(accs)
            u = [u_rows[i, pl.ds(j * L, L)] for j in range(NV)]
            for j in range(NV):
                accs[j] = accs[j] + u[j] * v_rows[i, pl.ds(j * L, L)]
            for n in range(NNEG):
                for j in range(NV):
                    k = (n + 1) * NV + j
                    accs[k] = accs[k] + u[j] * n_rows[n, i, pl.ds(j * L, L)]
            return tuple(accs)

        zero = jnp.zeros((L,), jnp.float32)
        accs = lax.fori_loop(0, bpw, body, (zero,) * (NT * NV))
        for t in range(NT):
            a = accs[t * NV:(t + 1) * NV]
            pout[pl.ds(t * L, L)] = (a[0] + a[1]) + (a[2] + a[3])
        pltpu.sync_copy(pout, out_hbm.at[wid])

    return sc_partials


def _tc_combine_body(p_ref, o_ref):
    x = p_ref[...]
    grp = lax.broadcasted_iota(jnp.int32, x.shape, 1) // L

    def logsig(z):
        return jnp.minimum(z, 0.0) - jnp.log1p(jnp.exp(-jnp.abs(z)))

    s = [jnp.sum(jnp.where(grp == t, x, 0.0)) for t in range(NT)]
    loss = -(logsig(s[0]) + sum(logsig(-s[t]) for t in range(1, NT)))
    o_ref[...] = loss * jnp.ones((1, 1), jnp.float32)


@jax.jit
def kernel(input_words, context_words, neg_words, u_emb, v_emb):
    B = input_words.shape[0]
    partials = _make_sc_partials(B)(
        input_words.astype(jnp.int32),
        context_words.astype(jnp.int32),
        neg_words.astype(jnp.int32).reshape(-1),
        u_emb,
        v_emb,
    )
    loss = pl.pallas_call(
        _tc_combine_body,
        out_shape=jax.ShapeDtypeStruct((1, 1), jnp.float32),
    )(partials)
    return loss[0, 0]


# speedup vs baseline: 1.5779x; 1.5779x over previous
import functools

import jax
import jax.numpy as jnp
from jax import lax
from jax.experimental import pallas as pl
from jax.experimental.pallas import tpu as pltpu
from jax.experimental.pallas import tpu_sc as plsc

D = 64
NNEG = 5
NC = 2
NS = 16
L = 16
NW = NC * NS
NV = D // L
NT = 1 + NNEG
PW = 128


@functools.lru_cache(maxsize=None)
def _make_sc_partials(B: int):
    assert B % NW == 0
    bpw = B // NW
    mesh = plsc.VectorSubcoreMesh(core_axis_name="c", subcore_axis_name="s")

    @functools.partial(
        pl.kernel,
        out_type=jax.ShapeDtypeStruct((NW, PW), jnp.float32),
        mesh=mesh,
        scratch_types=[
            pltpu.VMEM((NT + 1, bpw), jnp.int32),
            pltpu.VMEM((bpw, D), jnp.float32),
            pltpu.VMEM((bpw, D), jnp.float32),
            pltpu.VMEM((NNEG, bpw, D), jnp.float32),
            pltpu.VMEM((PW,), jnp.float32),
            pltpu.SemaphoreType.DMA,
        ],
    )
    def sc_partials(iw_hbm, cw_hbm, nw_hbm, u_hbm, v_hbm, out_hbm,
                    idx_vm, u_rows, v_rows, n_rows, pout, sem):
        wid = lax.axis_index("s") * NC + lax.axis_index("c")
        base = wid * bpw

        pltpu.sync_copy(iw_hbm.at[pl.ds(base, bpw)], idx_vm.at[0])
        pltpu.sync_copy(cw_hbm.at[pl.ds(base, bpw)], idx_vm.at[1])
        for n in range(NNEG):
            pltpu.sync_copy(nw_hbm.at[pl.ds(n * B + base, bpw)],
                            idx_vm.at[2 + n])

        def fire(g, carry):
            vu = idx_vm[0, pl.ds(g * L, L)]
            vc = idx_vm[1, pl.ds(g * L, L)]
            vns = [idx_vm[2 + n, pl.ds(g * L, L)] for n in range(NNEG)]
            for k in range(L):
                i = g * L + k
                pltpu.async_copy(u_hbm.at[pl.ds(vu[k], 1)],
                                 u_rows.at[pl.ds(i, 1)], sem)
                pltpu.async_copy(v_hbm.at[pl.ds(vc[k], 1)],
                                 v_rows.at[pl.ds(i, 1)], sem)
                for n in range(NNEG):
                    pltpu.async_copy(v_hbm.at[pl.ds(vns[n][k], 1)],
                                     n_rows.at[n].at[pl.ds(i, 1)], sem)
            return carry

        lax.fori_loop(0, bpw // L, fire, 0)

        pltpu.make_async_copy(u_hbm.at[pl.ds(0, bpw)], u_rows, sem).wait()
        pltpu.make_async_copy(v_hbm.at[pl.ds(0, bpw)], v_rows, sem).wait()
        for n in range(NNEG):
            pltpu.make_async_copy(v_hbm.at[pl.ds(0, bpw)], n_rows.at[n],
                                  sem).wait()

        def body(i, accs):
            accs = list(accs)
            u = [u_rows[i, pl.ds(j * L, L)] for j in range(NV)]
            for j in range(NV):
                accs[j] = accs[j] + u[j] * v_rows[i, pl.ds(j * L, L)]
            for n in range(NNEG):
                for j in range(NV):
                    k = (n + 1) * NV + j
                    accs[k] = accs[k] + u[j] * n_rows[n, i, pl.ds(j * L, L)]
            return tuple(accs)

        zero = jnp.zeros((L,), jnp.float32)
        accs = lax.fori_loop(0, bpw, body, (zero,) * (NT * NV))
        for t in range(NT):
            a = accs[t * NV:(t + 1) * NV]
            pout[pl.ds(t * L, L)] = (a[0] + a[1]) + (a[2] + a[3])
        for t in range(NT, PW // L):
            pout[pl.ds(t * L, L)] = zero
        pltpu.sync_copy(pout, out_hbm.at[wid])

    return sc_partials


def _tc_combine_body(p_ref, o_ref):
    x = p_ref[...]
    grp = lax.broadcasted_iota(jnp.int32, x.shape, 1) // L

    def logsig(z):
        return jnp.minimum(z, 0.0) - jnp.log1p(jnp.exp(-jnp.abs(z)))

    s = [jnp.sum(jnp.where(grp == t, x, 0.0)) for t in range(NT)]
    loss = -(logsig(s[0]) + sum(logsig(-s[t]) for t in range(1, NT)))
    o_ref[...] = loss * jnp.ones((1, 1), jnp.float32)


@jax.jit
def kernel(input_words, context_words, neg_words, u_emb, v_emb):
    B = input_words.shape[0]
    partials = _make_sc_partials(B)(
        input_words.astype(jnp.int32),
        context_words.astype(jnp.int32),
        neg_words.astype(jnp.int32).reshape(-1),
        u_emb,
        v_emb,
    )
    loss = pl.pallas_call(
        _tc_combine_body,
        out_shape=jax.ShapeDtypeStruct((1, 1), jnp.float32),
    )(partials)
    return loss[0, 0]
